# baseline (speedup 1.0000x reference)
import functools

import jax
import jax.numpy as jnp
from jax import lax
from jax.experimental import pallas as pl
from jax.experimental.pallas import tpu as pltpu
from jax.experimental.pallas import tpu_sc as plsc

F = 26
V = 1000
D = 16
B = 4096
NJ = F - 1

NC = 2
NS = 16
NW = NC * NS
PAIRS = F * NJ
P_IT = -(-PAIRS // NW)
BH = B // 2
QH = BH // 16
CH2 = BH // 128


def _make_kernel():
    mesh = plsc.VectorSubcoreMesh(core_axis_name="c", subcore_axis_name="s")

    @functools.partial(
        pl.kernel,
        mesh=mesh,
        out_type=jax.ShapeDtypeStruct((F, NJ, 2, B // 128, 8, 128), jnp.float32),
        compiler_params=pltpu.CompilerParams(
            use_tc_tiling_on_sc=False, needs_layout_passes=False
        ),
        scratch_types=[
            pltpu.VMEM((D, V), jnp.float32),
            pltpu.VMEM((D, V), jnp.float32),
            pltpu.VMEM((B,), jnp.int32),
            pltpu.VMEM((B,), jnp.int32),
            pltpu.VMEM((2, CH2, 8, 128), jnp.float32),
            pltpu.VMEM((2, CH2, 8, 128), jnp.float32),
            pltpu.SemaphoreType.DMA,
            pltpu.SemaphoreType.DMA,
        ],
    )
    def k(xt_hbm, tab_hbm, out_hbm, tbl0_v, tbl1_v, x0_v, x1_v,
          outa_v, outb_v, sem, sem2):
        c = lax.axis_index("c")
        s = lax.axis_index("s")
        w = s * NC + c

        def fill(out_ref, tbl_ref, x_ref, h):
            @plsc.parallel_loop(0, QH, 1, unroll=32)
            def _body(q):
                cc = q // 8
                c0 = (q % 8) * 16
                xv = x_ref[pl.ds(h * BH + q * 16, 16)]
                for d in range(D):
                    out_ref[d // 8, cc, d % 8, pl.ds(c0, 16)] = (
                        plsc.load_gather(tbl_ref.at[d], [xv])
                    )

        def wait_half(out_ref):
            pltpu.make_async_copy(
                out_hbm.at[0, 0, :, pl.ds(0, CH2)], out_ref, sem
            ).wait()

        def load_pair(p, tbl_ref, x_ref):
            i = p // NJ
            jj = p % NJ
            j = jj + jnp.where(jj >= i, 1, 0).astype(jj.dtype)
            pltpu.async_copy(tab_hbm.at[i, j], tbl_ref, sem2)
            pltpu.async_copy(xt_hbm.at[i], x_ref, sem2)

        def wait_pair(tbl_ref, x_ref):
            pltpu.make_async_copy(tab_hbm.at[0, 0], tbl_ref, sem2).wait()
            pltpu.make_async_copy(xt_hbm.at[0], x_ref, sem2).wait()

        def run_pair(p, tbl_ref, x_ref, wait_ab):
            i = p // NJ
            jj = p % NJ
            wait_pair(tbl_ref, x_ref)
            if wait_ab:
                wait_half(outa_v)
            fill(outa_v, tbl_ref, x_ref, 0)
            if wait_ab:
                wait_half(outb_v)
            pltpu.async_copy(
                outa_v, out_hbm.at[i, jj, :, pl.ds(0, CH2)], sem
            )
            fill(outb_v, tbl_ref, x_ref, 1)
            pltpu.async_copy(
                outb_v, out_hbm.at[i, jj, :, pl.ds(CH2, CH2)], sem
            )

        def prefetch(t, tbl_ref, x_ref):
            p = w + NW * t

            @pl.when(p < PAIRS)
            def _():
                load_pair(p, tbl_ref, x_ref)

        load_pair(w, tbl0_v, x0_v)
        prefetch(1, tbl1_v, x1_v)
        run_pair(w, tbl0_v, x0_v, False)

        def body(t2, carry):
            p1 = w + NW * (2 * t2 + 1)
            p2 = w + NW * (2 * t2 + 2)

            @pl.when(p1 < PAIRS)
            def _():
                prefetch(2 * t2 + 2, tbl0_v, x0_v)
                run_pair(p1, tbl1_v, x1_v, True)

            @pl.when(p2 < PAIRS)
            def _():
                prefetch(2 * t2 + 3, tbl1_v, x1_v)
                run_pair(p2, tbl0_v, x0_v, True)

            return carry

        lax.fori_loop(0, (P_IT - 1) // 2, body, 0)
        wait_half(outa_v)
        wait_half(outb_v)

    return k


_GATHER = _make_kernel()


def kernel(x, tables):
    tt = tables.transpose(0, 1, 3, 2)
    out6 = _GATHER(x.T, tt)
    out = out6.transpose(0, 1, 2, 4, 3, 5).reshape(F, NJ, D, B)
    return out.transpose(0, 3, 1, 2)


# speedup vs baseline: 1.1151x; 1.1151x over previous
import functools

import jax
import jax.numpy as jnp
from jax import lax
from jax.experimental import pallas as pl
from jax.experimental.pallas import tpu as pltpu
from jax.experimental.pallas import tpu_sc as plsc

F = 26
V = 1000
D = 16
B = 4096
NJ = F - 1

NC = 2
NS = 16
NW = NC * NS
PAIRS = F * NJ
P_IT = -(-PAIRS // NW)
BH = B // 2
QH = BH // 16
CH2 = BH // 128


def _make_kernel():
    mesh = plsc.VectorSubcoreMesh(core_axis_name="c", subcore_axis_name="s")

    @functools.partial(
        pl.kernel,
        mesh=mesh,
        out_type=jax.ShapeDtypeStruct((F, NJ, 2, B // 128, 8, 128), jnp.float32),
        compiler_params=pltpu.CompilerParams(
            use_tc_tiling_on_sc=False, needs_layout_passes=False
        ),
        scratch_types=[
            pltpu.VMEM((D, V), jnp.float32),
            pltpu.VMEM((D, V), jnp.float32),
            pltpu.VMEM((B,), jnp.int32),
            pltpu.VMEM((B,), jnp.int32),
            pltpu.VMEM((2, CH2, 8, 128), jnp.float32),
            pltpu.VMEM((2, CH2, 8, 128), jnp.float32),
            pltpu.SemaphoreType.DMA,
            pltpu.SemaphoreType.DMA,
        ],
    )
    def k(xt_hbm, tab_hbm, out_hbm, tbl0_v, tbl1_v, x0_v, x1_v,
          outa_v, outb_v, sem, sem2):
        c = lax.axis_index("c")
        s = lax.axis_index("s")
        w = s * NC + c

        def fill(out_ref, tbl_ref, x_ref, h):
            @plsc.parallel_loop(0, QH, 1, unroll=8)
            def _body(q):
                cc = q // 8
                c0 = (q % 8) * 16
                xv = x_ref[pl.ds(h * BH + q * 16, 16)]
                for d in range(D):
                    out_ref[d // 8, cc, d % 8, pl.ds(c0, 16)] = (
                        plsc.load_gather(tbl_ref.at[d], [xv])
                    )

        def wait_half(out_ref):
            pltpu.make_async_copy(
                out_hbm.at[0, 0, :, pl.ds(0, CH2)], out_ref, sem
            ).wait()

        def load_pair(p, tbl_ref, x_ref):
            i = p // NJ
            jj = p % NJ
            j = jj + jnp.where(jj >= i, 1, 0).astype(jj.dtype)
            pltpu.async_copy(tab_hbm.at[i, j], tbl_ref, sem2)
            pltpu.async_copy(xt_hbm.at[i], x_ref, sem2)

        def wait_pair(tbl_ref, x_ref):
            pltpu.make_async_copy(tab_hbm.at[0, 0], tbl_ref, sem2).wait()
            pltpu.make_async_copy(xt_hbm.at[0], x_ref, sem2).wait()

        def run_pair(p, tbl_ref, x_ref, wait_ab):
            i = p // NJ
            jj = p % NJ
            wait_pair(tbl_ref, x_ref)
            if wait_ab:
                wait_half(outa_v)
            fill(outa_v, tbl_ref, x_ref, 0)
            if wait_ab:
                wait_half(outb_v)
            pltpu.async_copy(
                outa_v, out_hbm.at[i, jj, :, pl.ds(0, CH2)], sem
            )
            fill(outb_v, tbl_ref, x_ref, 1)
            pltpu.async_copy(
                outb_v, out_hbm.at[i, jj, :, pl.ds(CH2, CH2)], sem
            )

        def prefetch(t, tbl_ref, x_ref):
            p = w + NW * t

            @pl.when(p < PAIRS)
            def _():
                load_pair(p, tbl_ref, x_ref)

        load_pair(w, tbl0_v, x0_v)
        prefetch(1, tbl1_v, x1_v)
        run_pair(w, tbl0_v, x0_v, False)

        def body(t2, carry):
            p1 = w + NW * (2 * t2 + 1)
            p2 = w + NW * (2 * t2 + 2)

            @pl.when(p1 < PAIRS)
            def _():
                prefetch(2 * t2 + 2, tbl0_v, x0_v)
                run_pair(p1, tbl1_v, x1_v, True)

            @pl.when(p2 < PAIRS)
            def _():
                prefetch(2 * t2 + 3, tbl1_v, x1_v)
                run_pair(p2, tbl0_v, x0_v, True)

            return carry

        lax.fori_loop(0, (P_IT - 1) // 2, body, 0)
        wait_half(outa_v)
        wait_half(outb_v)

    return k


_GATHER = _make_kernel()


def kernel(x, tables):
    tt = tables.transpose(0, 1, 3, 2)
    out6 = _GATHER(x.T, tt)
    out = out6.transpose(0, 1, 2, 4, 3, 5).reshape(F, NJ, D, B)
    return out.transpose(0, 3, 1, 2)
